# baseline (speedup 1.0000x reference)
import functools
import jax
import jax.numpy as jnp
from jax import lax
from jax.experimental import pallas as pl
from jax.experimental.pallas import tpu as pltpu
from jax.experimental.pallas import tpu_sc as plsc

B = 200
NPG = 250
M = 400
H = 128
N = B * NPG
E = 2 * M * B
NP = 50048
STRIPE = NP // 16
KB = 125
R = 2000

f32 = jnp.float32
i32 = jnp.int32


def _vmesh():
    return plsc.VectorSubcoreMesh(core_axis_name="c", subcore_axis_name="s")


_SC_PARAMS = pltpu.CompilerParams(use_tc_tiling_on_sc=False)


def sc_count(d12, zeros16, ones_src):
    @functools.partial(
        pl.kernel,
        out_type=jax.ShapeDtypeStruct((2, NP, 16), f32),
        mesh=_vmesh(),
        compiler_params=_SC_PARAMS,
        scratch_types=[pltpu.VMEM((40, KB), i32),
                       pltpu.VMEM((KB, 16), f32),
                       pltpu.VMEM_SHARED((NP, 16), f32)],
    )
    def k(d_hbm, z_hbm, ones_hbm, out_hbm, didx_v, ones_v, acc):
        ci = lax.axis_index("c")
        ti = lax.axis_index("s")
        pltpu.sync_copy(z_hbm, acc.at[pl.ds(ti * STRIPE, STRIPE)])
        pltpu.sync_copy(ones_hbm, ones_v)
        pltpu.sync_copy(d_hbm.at[ci, ti], didx_v)
        plsc.subcore_barrier()

        @pl.loop(0, 40)
        def _(j):
            pltpu.sync_copy(ones_v, acc.at[didx_v.at[j]], add=True)

        plsc.subcore_barrier()
        pltpu.sync_copy(acc.at[pl.ds(ti * STRIPE, STRIPE)],
                        out_hbm.at[ci, pl.ds(ti * STRIPE, STRIPE)])

    return k(d12, zeros16, ones_src)


def sc_scatter16(s12, d12, xp, zeros16):
    @functools.partial(
        pl.kernel,
        out_type=jax.ShapeDtypeStruct((2, NP, 16), f32),
        mesh=_vmesh(),
        compiler_params=_SC_PARAMS,
        scratch_types=[pltpu.VMEM((40, KB), i32),
                       pltpu.VMEM((40, KB), i32),
                       pltpu.VMEM((KB, 16), f32),
                       pltpu.VMEM((KB, 16), f32),
                       pltpu.VMEM_SHARED((NP, 16), f32),
                       pltpu.SemaphoreType.DMA,
                       pltpu.SemaphoreType.DMA],
    )
    def k(s_hbm, d_hbm, t_hbm, z_hbm, out_hbm, sidx_v, didx_v, rows0, rows1,
          acc, sem0, sem1):
        ci = lax.axis_index("c")
        ti = lax.axis_index("s")
        pltpu.sync_copy(z_hbm, acc.at[pl.ds(ti * STRIPE, STRIPE)])
        pltpu.sync_copy(s_hbm.at[ci, ti], sidx_v)
        pltpu.sync_copy(d_hbm.at[ci, ti], didx_v)
        plsc.subcore_barrier()
        pltpu.async_copy(t_hbm.at[sidx_v.at[0]], rows0, sem0)

        @pl.loop(0, 40, step=2)
        def _(j):
            pltpu.async_copy(t_hbm.at[sidx_v.at[j + 1]], rows1, sem1)
            pltpu.make_async_copy(t_hbm.at[sidx_v.at[0]], rows0, sem0).wait()
            pltpu.sync_copy(rows0, acc.at[didx_v.at[j]], add=True)

            @pl.when(j + 2 < 40)
            def _():
                pltpu.async_copy(t_hbm.at[sidx_v.at[j + 2]], rows0, sem0)

            pltpu.make_async_copy(t_hbm.at[sidx_v.at[0]], rows1, sem1).wait()
            pltpu.sync_copy(rows1, acc.at[didx_v.at[j + 1]], add=True)

        plsc.subcore_barrier()
        pltpu.sync_copy(acc.at[pl.ds(ti * STRIPE, STRIPE)],
                        out_hbm.at[ci, pl.ds(ti * STRIPE, STRIPE)])

    return k(s12, d12, xp, zeros16)


def sc_conv2(s3o, d3, hp_flat, zeros32):
    @functools.partial(
        pl.kernel,
        out_type=jax.ShapeDtypeStruct((2, 2, NP, 32), f32),
        mesh=_vmesh(),
        compiler_params=_SC_PARAMS,
        scratch_types=[pltpu.VMEM((80, KB), i32),
                       pltpu.VMEM((80, KB), i32),
                       pltpu.VMEM((KB, 32), f32),
                       pltpu.VMEM((KB, 32), f32),
                       pltpu.VMEM_SHARED((NP, 32), f32),
                       pltpu.SemaphoreType.DMA,
                       pltpu.SemaphoreType.DMA],
    )
    def k(s_hbm, d_hbm, t_hbm, z_hbm, out_hbm, sidx_v, didx_v, rows0, rows1,
          acc, sem0, sem1):
        ci = lax.axis_index("c")
        ti = lax.axis_index("s")
        pltpu.sync_copy(d_hbm.at[ti], didx_v)
        for kk in range(2):
            pltpu.sync_copy(s_hbm.at[ci, kk, ti], sidx_v)
            pltpu.sync_copy(z_hbm, acc.at[pl.ds(ti * STRIPE, STRIPE)])
            plsc.subcore_barrier()
            pltpu.async_copy(t_hbm.at[sidx_v.at[0]], rows0, sem0)

            @pl.loop(0, 80, step=2)
            def _(j):
                pltpu.async_copy(t_hbm.at[sidx_v.at[j + 1]], rows1, sem1)
                pltpu.make_async_copy(t_hbm.at[sidx_v.at[0]], rows0,
                                      sem0).wait()
                pltpu.sync_copy(rows0, acc.at[didx_v.at[j]], add=True)

                @pl.when(j + 2 < 80)
                def _():
                    pltpu.async_copy(t_hbm.at[sidx_v.at[j + 2]], rows0, sem0)

                pltpu.make_async_copy(t_hbm.at[sidx_v.at[0]], rows1,
                                      sem1).wait()
                pltpu.sync_copy(rows1, acc.at[didx_v.at[j + 1]], add=True)

            plsc.subcore_barrier()
            pltpu.sync_copy(acc.at[pl.ds(ti * STRIPE, STRIPE)],
                            out_hbm.at[ci, kk, pl.ds(ti * STRIPE, STRIPE)])

    return k(s3o, d3, hp_flat, zeros32)


def sc_gather(table, pcidx):
    @functools.partial(
        pl.kernel,
        out_type=jax.ShapeDtypeStruct((2, 80000, H), jnp.bfloat16),
        mesh=_vmesh(),
        compiler_params=_SC_PARAMS,
        scratch_types=[pltpu.VMEM((40, KB), i32),
                       pltpu.VMEM((KB, H), jnp.bfloat16),
                       pltpu.VMEM((KB, H), jnp.bfloat16),
                       pltpu.SemaphoreType.DMA,
                       pltpu.SemaphoreType.DMA],
    )
    def k(t_hbm, i_hbm, out_hbm, sidx_v, rows0, rows1, sem0, sem1):
        ci = lax.axis_index("c")
        ti = lax.axis_index("s")
        pltpu.sync_copy(i_hbm.at[ci, ti], sidx_v)
        base = ti * 5000
        pltpu.async_copy(t_hbm.at[sidx_v.at[0]], rows0, sem0)

        @pl.loop(0, 40, step=2)
        def _(j):
            pltpu.async_copy(t_hbm.at[sidx_v.at[j + 1]], rows1, sem1)
            pltpu.make_async_copy(t_hbm.at[sidx_v.at[0]], rows0, sem0).wait()
            pltpu.sync_copy(rows0, out_hbm.at[ci, pl.ds(base + j * KB, KB)])

            @pl.when(j + 2 < 40)
            def _():
                pltpu.async_copy(t_hbm.at[sidx_v.at[j + 2]], rows0, sem0)

            pltpu.make_async_copy(t_hbm.at[sidx_v.at[0]], rows1, sem1).wait()
            pltpu.sync_copy(rows1,
                            out_hbm.at[ci, pl.ds(base + (j + 1) * KB, KB)])

    return k(table, pcidx)


def tc_prep(cnt, x):
    def body(cnt_ref, x_ref, o_ref):
        c = cnt_ref[0, :, 0:1] + cnt_ref[1, :, 0:1]
        dis = lax.rsqrt(1.0 + c)
        xs = x_ref[...] * dis
        o_ref[...] = jnp.concatenate(
            [xs, dis, jnp.zeros((R, 13), f32)], axis=1)

    return pl.pallas_call(
        body,
        grid=(N // R,),
        in_specs=[pl.BlockSpec((2, R, 16), lambda i: (0, i, 0)),
                  pl.BlockSpec((R, 2), lambda i: (i, 0))],
        out_specs=pl.BlockSpec((R, 16), lambda i: (i, 0)),
        out_shape=jax.ShapeDtypeStruct((N, 16), f32),
    )(cnt, x)


def tc_conv1(scat1, xp, W_g1, b_g1):
    def body(s_ref, xp_ref, w_ref, b_ref, o_ref):
        tot = s_ref[0, :, 0:2] + s_ref[1, :, 0:2] + xp_ref[:, 0:2]
        dis = xp_ref[:, 2:3]
        agg = tot * dis
        h = jax.nn.relu(agg[:, 0:1] * w_ref[0:1, :]
                        + agg[:, 1:2] * w_ref[1:2, :] + b_ref[...])
        o_ref[...] = h * dis

    return pl.pallas_call(
        body,
        grid=(N // R,),
        in_specs=[pl.BlockSpec((2, R, 16), lambda i: (0, i, 0)),
                  pl.BlockSpec((R, 16), lambda i: (i, 0)),
                  pl.BlockSpec((2, H), lambda i: (0, 0)),
                  pl.BlockSpec((1, H), lambda i: (0, 0))],
        out_specs=pl.BlockSpec((R, H), lambda i: (i, 0)),
        out_shape=jax.ShapeDtypeStruct((N, H), f32),
    )(scat1, xp, W_g1, b_g1.reshape(1, H))


def tc_finalize(scat2, hp, xp, W_g2, b_g2):
    def body(s_ref, hp_ref, xp_ref, wg_ref, bg_ref, ag_ref, S_ref):
        full = jnp.concatenate([s_ref[c] for c in range(4)],
                               axis=1) + hp_ref[...]
        ag = full * xp_ref[:, 2:3]
        xg = lax.dot_general(ag.astype(jnp.bfloat16), wg_ref[...],
                             (((1,), (0,)), ((), ())),
                             preferred_element_type=f32) + bg_ref[...]
        agr = xg.reshape(R // NPG, NPG, H)
        ag_ref[...] = agr.astype(jnp.bfloat16)
        S_ref[...] = jnp.sum(agr, axis=1, keepdims=True)

    return pl.pallas_call(
        body,
        grid=(N // R,),
        in_specs=[pl.BlockSpec((4, R, 32), lambda i: (0, i, 0)),
                  pl.BlockSpec((R, H), lambda i: (i, 0)),
                  pl.BlockSpec((R, 16), lambda i: (i, 0)),
                  pl.BlockSpec((H, H), lambda i: (0, 0)),
                  pl.BlockSpec((1, H), lambda i: (0, 0))],
        out_specs=[pl.BlockSpec((R // NPG, NPG, H), lambda i: (i, 0, 0)),
                   pl.BlockSpec((R // NPG, 1, H), lambda i: (i, 0, 0))],
        out_shape=[jax.ShapeDtypeStruct((B, NPG, H), jnp.bfloat16),
                   jax.ShapeDtypeStruct((B, 1, H), f32)],
    )(scat2, hp, xp, W_g2.astype(jnp.bfloat16), b_g2.reshape(1, H))


def tc_main(x12, S, ns3, invd, incp, incc, A, AT, Ws1r, Wc1r,
            W_s2, b_s2, W_c2, b_c2, bs1, bc1, xiT):
    GB = 8
    MB = GB * M

    def body(x1_ref, x2_ref, S_ref, ns_ref, invd_ref, incp_ref, incc_ref,
             A_ref, AT_ref, Gs_ref, Gc_ref, Ws2_ref, bs2_ref, Wc2_ref,
             bc2_ref, bse_ref, bce_ref, xiT_ref, z_ref, zc_ref):
        dot = lambda a, b: lax.dot_general(
            a, b, (((1,), (0,)), ((), ())), preferred_element_type=f32)
        dotT = lambda a, b: lax.dot_general(
            a, b, (((1,), (1,)), ((), ())), preferred_element_type=f32)
        bf = jnp.bfloat16
        x1 = x1_ref[0]
        x2 = x2_ref[0]
        Sg = S_ref[:, 0, :].astype(bf)
        sg_term = dot(Sg, Gs_ref[2]) + bse_ref[...]
        cg_term = dot(Sg, Gc_ref[2]) + bce_ref[...]
        sg_full = jnp.repeat(sg_term, M, axis=0)
        cg_full = jnp.repeat(cg_term, M, axis=0)
        s_pre = dot(x1, Gs_ref[0]) + dot(x2, Gs_ref[1]) + sg_full
        sml = dot(jax.nn.relu(s_pre).astype(bf), Ws2_ref[...]) + bs2_ref[...]
        c_pre = dot(x1, Gc_ref[0]) + dot(x2, Gc_ref[1]) + cg_full
        cml = dot(jax.nn.relu(c_pre).astype(bf), Wc2_ref[...]) + bc2_ref[...]
        smlT = sml.T.reshape(4, GB, M)
        cmlT = cml.T.reshape(3, GB, M)
        ns = ns_ref[:, 0, :]
        jidx = lax.broadcasted_iota(i32, (GB, M), 1)
        mask = jidx >= (M - ns)
        one = jnp.ones((GB, M), f32)
        zero = jnp.zeros((GB, M), f32)
        graph_topo = jnp.where(mask, jax.nn.sigmoid(smlT[0]), one)
        p_flow = (jnp.where(mask, smlT[1], zero)
                  + jnp.where(mask, zero, cmlT[0]))
        vp = jnp.where(mask, smlT[2], zero) + jnp.where(mask, zero, cmlT[1])
        vcv = jnp.where(mask, smlT[3], zero) + jnp.where(mask, zero, cmlT[2])
        vpb = vp.astype(bf)
        vcb = vcv.astype(bf)
        vrows = [dotT(vpb[g:g + 1], incp_ref[g]) + dotT(vcb[g:g + 1],
                 incc_ref[g]) for g in range(GB)]
        vsum = jnp.concatenate(vrows, axis=0)
        v = invd_ref[:, 0, :] * vsum
        lidx = lax.broadcasted_iota(i32, (GB, NPG), 1)
        v = jnp.where(lidx == 0, jnp.float32(1.0), v)
        pfc = p_flow * graph_topo
        qfc = dot(v.astype(bf), A_ref[...]) * graph_topo
        pg = xiT_ref[:, 0, :] + dot(pfc.astype(bf), AT_ref[...])
        qg = xiT_ref[:, 1, :] + dot(qfc.astype(bf), AT_ref[...])
        z_ref[:, 0, 0:M] = pfc
        z_ref[:, 0, M:M + NPG] = v
        z_ref[:, 0, M + NPG:M + NPG + M] = graph_topo
        zc_ref[:, 0, 0:M] = qfc
        zc_ref[:, 0, M:M + NPG] = pg
        zc_ref[:, 0, M + NPG:M + 2 * NPG] = qg

    return pl.pallas_call(
        body,
        grid=(B // GB,),
        in_specs=[pl.BlockSpec((1, MB, H), lambda g: (0, g, 0)),
                  pl.BlockSpec((1, MB, H), lambda g: (1, g, 0)),
                  pl.BlockSpec((GB, 1, H), lambda g: (g, 0, 0)),
                  pl.BlockSpec((GB, 1, 1), lambda g: (g, 0, 0)),
                  pl.BlockSpec((GB, 1, NPG), lambda g: (g, 0, 0)),
                  pl.BlockSpec((GB, NPG, M), lambda g: (g, 0, 0)),
                  pl.BlockSpec((GB, NPG, M), lambda g: (g, 0, 0)),
                  pl.BlockSpec((NPG, M), lambda g: (0, 0)),
                  pl.BlockSpec((M, NPG), lambda g: (0, 0)),
                  pl.BlockSpec((3, H, 4 * H), lambda g: (0, 0, 0)),
                  pl.BlockSpec((3, H, 3 * H), lambda g: (0, 0, 0)),
                  pl.BlockSpec((4 * H, 4), lambda g: (0, 0)),
                  pl.BlockSpec((1, 4), lambda g: (0, 0)),
                  pl.BlockSpec((3 * H, 3), lambda g: (0, 0)),
                  pl.BlockSpec((1, 3), lambda g: (0, 0)),
                  pl.BlockSpec((1, 4 * H), lambda g: (0, 0)),
                  pl.BlockSpec((1, 3 * H), lambda g: (0, 0)),
                  pl.BlockSpec((GB, 2, NPG), lambda g: (g, 0, 0))],
        out_specs=[pl.BlockSpec((GB, 1, 2 * M + NPG), lambda g: (g, 0, 0)),
                   pl.BlockSpec((GB, 1, M + 2 * NPG), lambda g: (g, 0, 0))],
        out_shape=[jax.ShapeDtypeStruct((B, 1, 2 * M + NPG), f32),
                   jax.ShapeDtypeStruct((B, 1, M + 2 * NPG), f32)],
    )(x12, x12, S, ns3, invd, incp, incc, A, AT, Ws1r, Wc1r,
      W_s2, b_s2.reshape(1, 4), W_c2, b_c2.reshape(1, 3), bs1, bc1,
      xiT)


def kernel(x, edge_index, numSwitches, inv_degree, inc_parents, inc_childs, A,
           W_g1, b_g1, W_g2, b_g2, W_s1, b_s1, W_s2, b_s2, W_c1, b_c1,
           W_c2, b_c2):
    e0 = edge_index[0].astype(i32)
    e1 = edge_index[1].astype(i32)
    s12 = e0.reshape(2, 16, 40, KB)
    d12 = e1.reshape(2, 16, 40, KB)
    d3 = e1.reshape(16, 80, KB)
    cid = jnp.arange(4, dtype=i32).reshape(2, 2, 1, 1, 1)
    s3o = (e0 * 4).reshape(1, 1, 16, 80, KB) + cid
    er = e0.reshape(B, 2 * M)
    pcidx = jnp.stack([er[:, :M].reshape(-1),
                       er[:, M:].reshape(-1)]).reshape(2, 16, 40, KB)
    zeros16 = jnp.zeros((STRIPE, 16), f32)
    zeros32 = jnp.zeros((STRIPE, 32), f32)
    ones_src = jnp.ones((KB, 16), f32)

    cnt = sc_count(d12, zeros16, ones_src)
    xp = tc_prep(cnt[:, :N, :], x)
    scat1 = sc_scatter16(s12, d12, xp, zeros16)
    hp = tc_conv1(scat1[:, :N, :], xp, W_g1, b_g1)
    scat2 = sc_conv2(s3o, d3, hp.reshape(4 * N, 32), zeros32)
    ag, S = tc_finalize(scat2.reshape(4, NP, 32)[:, :N, :], hp, xp,
                        W_g2, b_g2)
    x12 = sc_gather(ag.reshape(N, H), pcidx)

    ns3 = numSwitches.astype(i32).reshape(B, 1, 1)
    invd = inv_degree.reshape(B, 1, NPG)
    xiT = x.reshape(B, NPG, 2).transpose(0, 2, 1)
    bf = jnp.bfloat16
    z, zc = tc_main(x12, S, ns3, invd, inc_parents.astype(bf),
                    inc_childs.astype(bf), A.astype(bf), A.T.astype(bf),
                    W_s1.reshape(3, H, 4 * H).astype(bf),
                    W_c1.reshape(3, H, 3 * H).astype(bf),
                    W_s2.astype(bf), b_s2, W_c2.astype(bf), b_c2,
                    b_s1.reshape(1, 4 * H), b_c1.reshape(1, 3 * H), xiT)
    return (z.reshape(B, 2 * M + NPG), zc.reshape(B, M + 2 * NPG))


# speedup vs baseline: 1.0275x; 1.0275x over previous
import functools
import jax
import jax.numpy as jnp
from jax import lax
from jax.experimental import pallas as pl
from jax.experimental.pallas import tpu as pltpu
from jax.experimental.pallas import tpu_sc as plsc

B = 200
NPG = 250
M = 400
H = 128
N = B * NPG
E = 2 * M * B
NP = 50048
STRIPE = NP // 16
KB = 125
R = 2000

f32 = jnp.float32
i32 = jnp.int32


def _vmesh():
    return plsc.VectorSubcoreMesh(core_axis_name="c", subcore_axis_name="s")


_SC_PARAMS = pltpu.CompilerParams(use_tc_tiling_on_sc=False)


def sc_count(d12, zeros16, ones_src):
    @functools.partial(
        pl.kernel,
        out_type=jax.ShapeDtypeStruct((2, NP, 16), f32),
        mesh=_vmesh(),
        compiler_params=_SC_PARAMS,
        scratch_types=[pltpu.VMEM((40, KB), i32),
                       pltpu.VMEM((KB, 16), f32),
                       pltpu.VMEM_SHARED((NP, 16), f32)],
    )
    def k(d_hbm, z_hbm, ones_hbm, out_hbm, didx_v, ones_v, acc):
        ci = lax.axis_index("c")
        ti = lax.axis_index("s")
        pltpu.sync_copy(z_hbm, acc.at[pl.ds(ti * STRIPE, STRIPE)])
        pltpu.sync_copy(ones_hbm, ones_v)
        pltpu.sync_copy(d_hbm.at[ci, ti], didx_v)
        plsc.subcore_barrier()

        @pl.loop(0, 40)
        def _(j):
            pltpu.sync_copy(ones_v, acc.at[didx_v.at[j]], add=True)

        plsc.subcore_barrier()
        pltpu.sync_copy(acc.at[pl.ds(ti * STRIPE, STRIPE)],
                        out_hbm.at[ci, pl.ds(ti * STRIPE, STRIPE)])

    return k(d12, zeros16, ones_src)


def sc_scatter16(s12, d12, xp, zeros16):
    @functools.partial(
        pl.kernel,
        out_type=jax.ShapeDtypeStruct((2, NP, 16), f32),
        mesh=_vmesh(),
        compiler_params=_SC_PARAMS,
        scratch_types=[pltpu.VMEM((40, KB), i32),
                       pltpu.VMEM((40, KB), i32),
                       pltpu.VMEM((KB, 16), f32),
                       pltpu.VMEM((KB, 16), f32),
                       pltpu.VMEM_SHARED((NP, 16), f32),
                       pltpu.SemaphoreType.DMA,
                       pltpu.SemaphoreType.DMA],
    )
    def k(s_hbm, d_hbm, t_hbm, z_hbm, out_hbm, sidx_v, didx_v, rows0, rows1,
          acc, sem0, sem1):
        ci = lax.axis_index("c")
        ti = lax.axis_index("s")
        pltpu.sync_copy(z_hbm, acc.at[pl.ds(ti * STRIPE, STRIPE)])
        pltpu.sync_copy(s_hbm.at[ci, ti], sidx_v)
        pltpu.sync_copy(d_hbm.at[ci, ti], didx_v)
        plsc.subcore_barrier()
        pltpu.async_copy(t_hbm.at[sidx_v.at[0]], rows0, sem0)

        @pl.loop(0, 40, step=2)
        def _(j):
            pltpu.async_copy(t_hbm.at[sidx_v.at[j + 1]], rows1, sem1)
            pltpu.make_async_copy(t_hbm.at[sidx_v.at[0]], rows0, sem0).wait()
            pltpu.sync_copy(rows0, acc.at[didx_v.at[j]], add=True)

            @pl.when(j + 2 < 40)
            def _():
                pltpu.async_copy(t_hbm.at[sidx_v.at[j + 2]], rows0, sem0)

            pltpu.make_async_copy(t_hbm.at[sidx_v.at[0]], rows1, sem1).wait()
            pltpu.sync_copy(rows1, acc.at[didx_v.at[j + 1]], add=True)

        plsc.subcore_barrier()
        pltpu.sync_copy(acc.at[pl.ds(ti * STRIPE, STRIPE)],
                        out_hbm.at[ci, pl.ds(ti * STRIPE, STRIPE)])

    return k(s12, d12, xp, zeros16)


def sc_conv2(s3o, d3, hp_flat, zeros32):
    @functools.partial(
        pl.kernel,
        out_type=jax.ShapeDtypeStruct((2, 2, NP, 32), f32),
        mesh=_vmesh(),
        compiler_params=_SC_PARAMS,
        scratch_types=[pltpu.VMEM((80, KB), i32),
                       pltpu.VMEM((80, KB), i32),
                       pltpu.VMEM((KB, 32), f32),
                       pltpu.VMEM((KB, 32), f32),
                       pltpu.VMEM_SHARED((NP, 32), f32),
                       pltpu.SemaphoreType.DMA,
                       pltpu.SemaphoreType.DMA],
    )
    def k(s_hbm, d_hbm, t_hbm, z_hbm, out_hbm, sidx_v, didx_v, rows0, rows1,
          acc, sem0, sem1):
        ci = lax.axis_index("c")
        ti = lax.axis_index("s")
        pltpu.sync_copy(d_hbm.at[ti], didx_v)
        for kk in range(2):
            pltpu.sync_copy(s_hbm.at[ci, kk, ti], sidx_v)
            pltpu.sync_copy(z_hbm, acc.at[pl.ds(ti * STRIPE, STRIPE)])
            plsc.subcore_barrier()
            pltpu.async_copy(t_hbm.at[sidx_v.at[0]], rows0, sem0)

            @pl.loop(0, 80, step=2)
            def _(j):
                pltpu.async_copy(t_hbm.at[sidx_v.at[j + 1]], rows1, sem1)
                pltpu.make_async_copy(t_hbm.at[sidx_v.at[0]], rows0,
                                      sem0).wait()
                pltpu.sync_copy(rows0, acc.at[didx_v.at[j]], add=True)

                @pl.when(j + 2 < 80)
                def _():
                    pltpu.async_copy(t_hbm.at[sidx_v.at[j + 2]], rows0, sem0)

                pltpu.make_async_copy(t_hbm.at[sidx_v.at[0]], rows1,
                                      sem1).wait()
                pltpu.sync_copy(rows1, acc.at[didx_v.at[j + 1]], add=True)

            plsc.subcore_barrier()
            pltpu.sync_copy(acc.at[pl.ds(ti * STRIPE, STRIPE)],
                            out_hbm.at[ci, kk, pl.ds(ti * STRIPE, STRIPE)])

    return k(s3o, d3, hp_flat, zeros32)


def sc_gather(table, pcidx):
    @functools.partial(
        pl.kernel,
        out_type=jax.ShapeDtypeStruct((2, 80000, H), jnp.bfloat16),
        mesh=_vmesh(),
        compiler_params=_SC_PARAMS,
        scratch_types=[pltpu.VMEM((40, KB), i32),
                       pltpu.VMEM((KB, H), jnp.bfloat16),
                       pltpu.VMEM((KB, H), jnp.bfloat16),
                       pltpu.SemaphoreType.DMA,
                       pltpu.SemaphoreType.DMA],
    )
    def k(t_hbm, i_hbm, out_hbm, sidx_v, rows0, rows1, sem0, sem1):
        ci = lax.axis_index("c")
        ti = lax.axis_index("s")
        pltpu.sync_copy(i_hbm.at[ci, ti], sidx_v)
        base = ti * 5000
        pltpu.async_copy(t_hbm.at[sidx_v.at[0]], rows0, sem0)

        @pl.loop(0, 40, step=2)
        def _(j):
            pltpu.async_copy(t_hbm.at[sidx_v.at[j + 1]], rows1, sem1)
            pltpu.make_async_copy(t_hbm.at[sidx_v.at[0]], rows0, sem0).wait()
            pltpu.sync_copy(rows0, out_hbm.at[ci, pl.ds(base + j * KB, KB)])

            @pl.when(j + 2 < 40)
            def _():
                pltpu.async_copy(t_hbm.at[sidx_v.at[j + 2]], rows0, sem0)

            pltpu.make_async_copy(t_hbm.at[sidx_v.at[0]], rows1, sem1).wait()
            pltpu.sync_copy(rows1,
                            out_hbm.at[ci, pl.ds(base + (j + 1) * KB, KB)])

    return k(table, pcidx)


def tc_prep(cnt, x):
    def body(cnt_ref, x_ref, o_ref):
        c = cnt_ref[0, :, 0:1] + cnt_ref[1, :, 0:1]
        dis = lax.rsqrt(1.0 + c)
        xs = x_ref[...] * dis
        o_ref[...] = jnp.concatenate(
            [xs, dis, jnp.zeros((R, 13), f32)], axis=1)

    return pl.pallas_call(
        body,
        grid=(N // R,),
        in_specs=[pl.BlockSpec((2, R, 16), lambda i: (0, i, 0)),
                  pl.BlockSpec((R, 2), lambda i: (i, 0))],
        out_specs=pl.BlockSpec((R, 16), lambda i: (i, 0)),
        out_shape=jax.ShapeDtypeStruct((N, 16), f32),
    )(cnt, x)


def tc_conv1(scat1, xp, W_g1, b_g1):
    def body(s_ref, xp_ref, w_ref, b_ref, o_ref):
        tot = s_ref[0, :, 0:2] + s_ref[1, :, 0:2] + xp_ref[:, 0:2]
        dis = xp_ref[:, 2:3]
        agg = tot * dis
        h = jax.nn.relu(agg[:, 0:1] * w_ref[0:1, :]
                        + agg[:, 1:2] * w_ref[1:2, :] + b_ref[...])
        o_ref[...] = h * dis

    return pl.pallas_call(
        body,
        grid=(N // R,),
        in_specs=[pl.BlockSpec((2, R, 16), lambda i: (0, i, 0)),
                  pl.BlockSpec((R, 16), lambda i: (i, 0)),
                  pl.BlockSpec((2, H), lambda i: (0, 0)),
                  pl.BlockSpec((1, H), lambda i: (0, 0))],
        out_specs=pl.BlockSpec((R, H), lambda i: (i, 0)),
        out_shape=jax.ShapeDtypeStruct((N, H), f32),
    )(scat1, xp, W_g1, b_g1.reshape(1, H))


def tc_finalize(scat2, hp, xp, W_g2, b_g2):
    def body(s_ref, hp_ref, xp_ref, wg_ref, bg_ref, ag_ref, S_ref):
        full = jnp.concatenate([s_ref[c] for c in range(4)],
                               axis=1) + hp_ref[...]
        ag = full * xp_ref[:, 2:3]
        xg = lax.dot_general(ag.astype(jnp.bfloat16), wg_ref[...],
                             (((1,), (0,)), ((), ())),
                             preferred_element_type=f32) + bg_ref[...]
        agr = xg.reshape(R // NPG, NPG, H)
        ag_ref[...] = agr.astype(jnp.bfloat16)
        S_ref[...] = jnp.sum(agr, axis=1, keepdims=True)

    return pl.pallas_call(
        body,
        grid=(N // R,),
        in_specs=[pl.BlockSpec((4, R, 32), lambda i: (0, i, 0)),
                  pl.BlockSpec((R, H), lambda i: (i, 0)),
                  pl.BlockSpec((R, 16), lambda i: (i, 0)),
                  pl.BlockSpec((H, H), lambda i: (0, 0)),
                  pl.BlockSpec((1, H), lambda i: (0, 0))],
        out_specs=[pl.BlockSpec((R // NPG, NPG, H), lambda i: (i, 0, 0)),
                   pl.BlockSpec((R // NPG, 1, H), lambda i: (i, 0, 0))],
        out_shape=[jax.ShapeDtypeStruct((B, NPG, H), jnp.bfloat16),
                   jax.ShapeDtypeStruct((B, 1, H), f32)],
    )(scat2, hp, xp, W_g2.astype(jnp.bfloat16), b_g2.reshape(1, H))


def tc_main(x12, S, ns3, invd, incp, incc, A, AT, Ws1r, Wc1r,
            W_s2, b_s2, W_c2, b_c2, bs1, bc1, xiT):
    GB = 8
    MB = GB * M

    def body(x1_ref, x2_ref, S_ref, ns_ref, invd_ref, incp_ref, incc_ref,
             A_ref, AT_ref, Gs_ref, Gc_ref, Ws2_ref, bs2_ref, Wc2_ref,
             bc2_ref, bse_ref, bce_ref, xiT_ref, z_ref, zc_ref):
        dot = lambda a, b: lax.dot_general(
            a, b, (((1,), (0,)), ((), ())), preferred_element_type=f32)
        dotT = lambda a, b: lax.dot_general(
            a, b, (((1,), (1,)), ((), ())), preferred_element_type=f32)
        bf = jnp.bfloat16
        x1 = x1_ref[0]
        x2 = x2_ref[0]
        Sg = S_ref[:, 0, :].astype(bf)
        sg_term = dot(Sg, Gs_ref[2]) + bse_ref[...]
        cg_term = dot(Sg, Gc_ref[2]) + bce_ref[...]
        sg_full = jnp.repeat(sg_term, M, axis=0)
        cg_full = jnp.repeat(cg_term, M, axis=0)
        s_pre = dot(x1, Gs_ref[0]) + dot(x2, Gs_ref[1]) + sg_full
        sml = dot(jax.nn.relu(s_pre).astype(bf), Ws2_ref[...]) + bs2_ref[...]
        c_pre = dot(x1, Gc_ref[0]) + dot(x2, Gc_ref[1]) + cg_full
        cml = dot(jax.nn.relu(c_pre).astype(bf), Wc2_ref[...]) + bc2_ref[...]
        smlT = sml.T.reshape(4, GB, M)
        cmlT = cml.T.reshape(3, GB, M)
        ns = ns_ref[:, 0, :]
        jidx = lax.broadcasted_iota(i32, (GB, M), 1)
        mask = jidx >= (M - ns)
        one = jnp.ones((GB, M), f32)
        zero = jnp.zeros((GB, M), f32)
        graph_topo = jnp.where(mask, jax.nn.sigmoid(smlT[0]), one)
        p_flow = (jnp.where(mask, smlT[1], zero)
                  + jnp.where(mask, zero, cmlT[0]))
        vp = jnp.where(mask, smlT[2], zero) + jnp.where(mask, zero, cmlT[1])
        vcv = jnp.where(mask, smlT[3], zero) + jnp.where(mask, zero, cmlT[2])
        vrows = [dotT(vp[g:g + 1], incp_ref[g]) + dotT(vcv[g:g + 1],
                 incc_ref[g]) for g in range(GB)]
        vsum = jnp.concatenate(vrows, axis=0)
        v = invd_ref[:, 0, :] * vsum
        lidx = lax.broadcasted_iota(i32, (GB, NPG), 1)
        v = jnp.where(lidx == 0, jnp.float32(1.0), v)
        pfc = p_flow * graph_topo
        qfc = dot(v, A_ref[...]) * graph_topo
        pg = xiT_ref[:, 0, :] + dot(pfc, AT_ref[...])
        qg = xiT_ref[:, 1, :] + dot(qfc, AT_ref[...])
        z_ref[:, 0, 0:M] = pfc
        z_ref[:, 0, M:M + NPG] = v
        z_ref[:, 0, M + NPG:M + NPG + M] = graph_topo
        zc_ref[:, 0, 0:M] = qfc
        zc_ref[:, 0, M:M + NPG] = pg
        zc_ref[:, 0, M + NPG:M + 2 * NPG] = qg

    return pl.pallas_call(
        body,
        grid=(B // GB,),
        in_specs=[pl.BlockSpec((1, MB, H), lambda g: (0, g, 0)),
                  pl.BlockSpec((1, MB, H), lambda g: (1, g, 0)),
                  pl.BlockSpec((GB, 1, H), lambda g: (g, 0, 0)),
                  pl.BlockSpec((GB, 1, 1), lambda g: (g, 0, 0)),
                  pl.BlockSpec((GB, 1, NPG), lambda g: (g, 0, 0)),
                  pl.BlockSpec((GB, NPG, M), lambda g: (g, 0, 0)),
                  pl.BlockSpec((GB, NPG, M), lambda g: (g, 0, 0)),
                  pl.BlockSpec((NPG, M), lambda g: (0, 0)),
                  pl.BlockSpec((M, NPG), lambda g: (0, 0)),
                  pl.BlockSpec((3, H, 4 * H), lambda g: (0, 0, 0)),
                  pl.BlockSpec((3, H, 3 * H), lambda g: (0, 0, 0)),
                  pl.BlockSpec((4 * H, 4), lambda g: (0, 0)),
                  pl.BlockSpec((1, 4), lambda g: (0, 0)),
                  pl.BlockSpec((3 * H, 3), lambda g: (0, 0)),
                  pl.BlockSpec((1, 3), lambda g: (0, 0)),
                  pl.BlockSpec((1, 4 * H), lambda g: (0, 0)),
                  pl.BlockSpec((1, 3 * H), lambda g: (0, 0)),
                  pl.BlockSpec((GB, 2, NPG), lambda g: (g, 0, 0))],
        out_specs=[pl.BlockSpec((GB, 1, 2 * M + NPG), lambda g: (g, 0, 0)),
                   pl.BlockSpec((GB, 1, M + 2 * NPG), lambda g: (g, 0, 0))],
        out_shape=[jax.ShapeDtypeStruct((B, 1, 2 * M + NPG), f32),
                   jax.ShapeDtypeStruct((B, 1, M + 2 * NPG), f32)],
    )(x12, x12, S, ns3, invd, incp, incc, A, AT, Ws1r, Wc1r,
      W_s2, b_s2.reshape(1, 4), W_c2, b_c2.reshape(1, 3), bs1, bc1,
      xiT)


def kernel(x, edge_index, numSwitches, inv_degree, inc_parents, inc_childs, A,
           W_g1, b_g1, W_g2, b_g2, W_s1, b_s1, W_s2, b_s2, W_c1, b_c1,
           W_c2, b_c2):
    e0 = edge_index[0].astype(i32)
    e1 = edge_index[1].astype(i32)
    s12 = e0.reshape(2, 16, 40, KB)
    d12 = e1.reshape(2, 16, 40, KB)
    d3 = e1.reshape(16, 80, KB)
    cid = jnp.arange(4, dtype=i32).reshape(2, 2, 1, 1, 1)
    s3o = (e0 * 4).reshape(1, 1, 16, 80, KB) + cid
    er = e0.reshape(B, 2 * M)
    pcidx = jnp.stack([er[:, :M].reshape(-1),
                       er[:, M:].reshape(-1)]).reshape(2, 16, 40, KB)
    zeros16 = jnp.zeros((STRIPE, 16), f32)
    zeros32 = jnp.zeros((STRIPE, 32), f32)
    ones_src = jnp.ones((KB, 16), f32)

    cnt = sc_count(d12, zeros16, ones_src)
    xp = tc_prep(cnt[:, :N, :], x)
    scat1 = sc_scatter16(s12, d12, xp, zeros16)
    hp = tc_conv1(scat1[:, :N, :], xp, W_g1, b_g1)
    scat2 = sc_conv2(s3o, d3, hp.reshape(4 * N, 32), zeros32)
    ag, S = tc_finalize(scat2.reshape(4, NP, 32)[:, :N, :], hp, xp,
                        W_g2, b_g2)
    x12 = sc_gather(ag.reshape(N, H), pcidx)

    ns3 = numSwitches.astype(i32).reshape(B, 1, 1)
    invd = inv_degree.reshape(B, 1, NPG)
    xiT = x.reshape(B, NPG, 2).transpose(0, 2, 1)
    bf = jnp.bfloat16
    z, zc = tc_main(x12, S, ns3, invd, inc_parents, inc_childs, A, A.T,
                    W_s1.reshape(3, H, 4 * H).astype(bf),
                    W_c1.reshape(3, H, 3 * H).astype(bf),
                    W_s2.astype(bf), b_s2, W_c2.astype(bf), b_c2,
                    b_s1.reshape(1, 4 * H), b_c1.reshape(1, 3 * H), xiT)
    return (z.reshape(B, 2 * M + NPG), zc.reshape(B, M + 2 * NPG))


# speedup vs baseline: 1.2528x; 1.2193x over previous
import functools
import jax
import jax.numpy as jnp
from jax import lax
from jax.experimental import pallas as pl
from jax.experimental.pallas import tpu as pltpu
from jax.experimental.pallas import tpu_sc as plsc

B = 200
NPG = 250
M = 400
H = 128
N = B * NPG
E = 2 * M * B
NP = 50048
STRIPE = NP // 16
KB = 125
R = 2000

f32 = jnp.float32
i32 = jnp.int32


def _vmesh():
    return plsc.VectorSubcoreMesh(core_axis_name="c", subcore_axis_name="s")


_SC_PARAMS = pltpu.CompilerParams(use_tc_tiling_on_sc=False)


def sc_count(d12, zeros16, ones_src):
    @functools.partial(
        pl.kernel,
        out_type=jax.ShapeDtypeStruct((2, NP, 16), f32),
        mesh=_vmesh(),
        compiler_params=_SC_PARAMS,
        scratch_types=[pltpu.VMEM((40, KB), i32),
                       pltpu.VMEM((KB, 16), f32),
                       pltpu.VMEM_SHARED((NP, 16), f32)],
    )
    def k(d_hbm, z_hbm, ones_hbm, out_hbm, didx_v, ones_v, acc):
        ci = lax.axis_index("c")
        ti = lax.axis_index("s")
        pltpu.sync_copy(z_hbm, acc.at[pl.ds(ti * STRIPE, STRIPE)])
        pltpu.sync_copy(ones_hbm, ones_v)
        pltpu.sync_copy(d_hbm.at[ci, ti], didx_v)
        plsc.subcore_barrier()

        @pl.loop(0, 40)
        def _(j):
            pltpu.sync_copy(ones_v, acc.at[didx_v.at[j]], add=True)

        plsc.subcore_barrier()
        pltpu.sync_copy(acc.at[pl.ds(ti * STRIPE, STRIPE)],
                        out_hbm.at[ci, pl.ds(ti * STRIPE, STRIPE)])

    return k(d12, zeros16, ones_src)


def sc_scatter16(s12, d12, xp, zeros16):
    @functools.partial(
        pl.kernel,
        out_type=jax.ShapeDtypeStruct((2, NP, 16), f32),
        mesh=_vmesh(),
        compiler_params=_SC_PARAMS,
        scratch_types=[pltpu.VMEM((40, KB), i32),
                       pltpu.VMEM((40, KB), i32),
                       pltpu.VMEM((KB, 16), f32),
                       pltpu.VMEM((KB, 16), f32),
                       pltpu.VMEM_SHARED((NP, 16), f32),
                       pltpu.SemaphoreType.DMA,
                       pltpu.SemaphoreType.DMA],
    )
    def k(s_hbm, d_hbm, t_hbm, z_hbm, out_hbm, sidx_v, didx_v, rows0, rows1,
          acc, sem0, sem1):
        ci = lax.axis_index("c")
        ti = lax.axis_index("s")
        pltpu.sync_copy(z_hbm, acc.at[pl.ds(ti * STRIPE, STRIPE)])
        pltpu.sync_copy(s_hbm.at[ci, ti], sidx_v)
        pltpu.sync_copy(d_hbm.at[ci, ti], didx_v)
        plsc.subcore_barrier()
        pltpu.async_copy(t_hbm.at[sidx_v.at[0]], rows0, sem0)

        @pl.loop(0, 40, step=2)
        def _(j):
            pltpu.async_copy(t_hbm.at[sidx_v.at[j + 1]], rows1, sem1)
            pltpu.make_async_copy(t_hbm.at[sidx_v.at[0]], rows0, sem0).wait()
            pltpu.sync_copy(rows0, acc.at[didx_v.at[j]], add=True)

            @pl.when(j + 2 < 40)
            def _():
                pltpu.async_copy(t_hbm.at[sidx_v.at[j + 2]], rows0, sem0)

            pltpu.make_async_copy(t_hbm.at[sidx_v.at[0]], rows1, sem1).wait()
            pltpu.sync_copy(rows1, acc.at[didx_v.at[j + 1]], add=True)

        plsc.subcore_barrier()
        pltpu.sync_copy(acc.at[pl.ds(ti * STRIPE, STRIPE)],
                        out_hbm.at[ci, pl.ds(ti * STRIPE, STRIPE)])

    return k(s12, d12, xp, zeros16)


def sc_conv2(s3, d3, hp4, zeros32):
    @functools.partial(
        pl.kernel,
        out_type=jax.ShapeDtypeStruct((2, 2, NP, 32), f32),
        mesh=_vmesh(),
        compiler_params=_SC_PARAMS,
        scratch_types=[pltpu.VMEM((80, KB), i32),
                       pltpu.VMEM((80, KB), i32),
                       pltpu.VMEM((KB, 32), f32),
                       pltpu.VMEM((KB, 32), f32),
                       pltpu.VMEM_SHARED((NP, 32), f32),
                       pltpu.SemaphoreType.DMA,
                       pltpu.SemaphoreType.DMA],
    )
    def k(s_hbm, d_hbm, t_hbm, z_hbm, out_hbm, sidx_v, didx_v, rows0, rows1,
          acc, sem0, sem1):
        ci = lax.axis_index("c")
        ti = lax.axis_index("s")
        pltpu.sync_copy(d_hbm.at[ti], didx_v)
        pltpu.sync_copy(s_hbm.at[ti], sidx_v)
        for cc in range(4):
            t_c = t_hbm.at[cc]

            @pl.when(ci == cc // 2)
            def _():
                kk = cc % 2
                pltpu.sync_copy(z_hbm, acc.at[pl.ds(ti * STRIPE, STRIPE)])
                plsc.subcore_barrier()
                pltpu.async_copy(t_c.at[sidx_v.at[0]], rows0, sem0)

                @pl.loop(0, 80, step=2)
                def _(j):
                    pltpu.async_copy(t_c.at[sidx_v.at[j + 1]], rows1, sem1)
                    pltpu.make_async_copy(t_c.at[sidx_v.at[0]], rows0,
                                          sem0).wait()
                    pltpu.sync_copy(rows0, acc.at[didx_v.at[j]], add=True)

                    @pl.when(j + 2 < 80)
                    def _():
                        pltpu.async_copy(t_c.at[sidx_v.at[j + 2]], rows0,
                                         sem0)

                    pltpu.make_async_copy(t_c.at[sidx_v.at[0]], rows1,
                                          sem1).wait()
                    pltpu.sync_copy(rows1, acc.at[didx_v.at[j + 1]], add=True)

                plsc.subcore_barrier()
                pltpu.sync_copy(acc.at[pl.ds(ti * STRIPE, STRIPE)],
                                out_hbm.at[ci, kk,
                                           pl.ds(ti * STRIPE, STRIPE)])

    return k(s3, d3, hp4, zeros32)


def sc_gather(table, pcidx):
    @functools.partial(
        pl.kernel,
        out_type=jax.ShapeDtypeStruct((2, 80000, H), f32),
        mesh=_vmesh(),
        compiler_params=_SC_PARAMS,
        scratch_types=[pltpu.VMEM((40, KB), i32),
                       pltpu.VMEM((KB, H), f32),
                       pltpu.VMEM((KB, H), f32),
                       pltpu.SemaphoreType.DMA,
                       pltpu.SemaphoreType.DMA],
    )
    def k(t_hbm, i_hbm, out_hbm, sidx_v, rows0, rows1, sem0, sem1):
        ci = lax.axis_index("c")
        ti = lax.axis_index("s")
        pltpu.sync_copy(i_hbm.at[ci, ti], sidx_v)
        base = ti * 5000
        pltpu.async_copy(t_hbm.at[sidx_v.at[0]], rows0, sem0)

        @pl.loop(0, 40, step=2)
        def _(j):
            pltpu.async_copy(t_hbm.at[sidx_v.at[j + 1]], rows1, sem1)
            pltpu.make_async_copy(t_hbm.at[sidx_v.at[0]], rows0, sem0).wait()
            pltpu.sync_copy(rows0, out_hbm.at[ci, pl.ds(base + j * KB, KB)])

            @pl.when(j + 2 < 40)
            def _():
                pltpu.async_copy(t_hbm.at[sidx_v.at[j + 2]], rows0, sem0)

            pltpu.make_async_copy(t_hbm.at[sidx_v.at[0]], rows1, sem1).wait()
            pltpu.sync_copy(rows1,
                            out_hbm.at[ci, pl.ds(base + (j + 1) * KB, KB)])

    return k(table, pcidx)


def tc_prep(cnt, x):
    def body(cnt_ref, x_ref, o_ref):
        c = cnt_ref[0, :, 0:1] + cnt_ref[1, :, 0:1]
        dis = lax.rsqrt(1.0 + c)
        xs = x_ref[...] * dis
        o_ref[...] = jnp.concatenate(
            [xs, dis, jnp.zeros((R, 13), f32)], axis=1)

    return pl.pallas_call(
        body,
        grid=(N // R,),
        in_specs=[pl.BlockSpec((2, R, 16), lambda i: (0, i, 0)),
                  pl.BlockSpec((R, 2), lambda i: (i, 0))],
        out_specs=pl.BlockSpec((R, 16), lambda i: (i, 0)),
        out_shape=jax.ShapeDtypeStruct((N, 16), f32),
    )(cnt, x)


def tc_conv1(scat1, xp, W_g1, b_g1):
    def body(s_ref, xp_ref, w_ref, b_ref, o_ref):
        tot = s_ref[0, :, 0:2] + s_ref[1, :, 0:2] + xp_ref[:, 0:2]
        dis = xp_ref[:, 2:3]
        agg = tot * dis
        h = jax.nn.relu(agg[:, 0:1] * w_ref[0:1, :]
                        + agg[:, 1:2] * w_ref[1:2, :] + b_ref[...])
        hp = h * dis
        o_ref[...] = jnp.stack([hp[:, 32 * c:32 * c + 32] for c in range(4)],
                               axis=0)

    return pl.pallas_call(
        body,
        grid=(N // R,),
        in_specs=[pl.BlockSpec((2, R, 16), lambda i: (0, i, 0)),
                  pl.BlockSpec((R, 16), lambda i: (i, 0)),
                  pl.BlockSpec((2, H), lambda i: (0, 0)),
                  pl.BlockSpec((1, H), lambda i: (0, 0))],
        out_specs=pl.BlockSpec((4, R, 32), lambda i: (0, i, 0)),
        out_shape=jax.ShapeDtypeStruct((4, N, 32), f32),
    )(scat1, xp, W_g1, b_g1.reshape(1, H))


def tc_finalize(scat2, hp, xp, W_g2, b_g2):
    def body(s_ref, hp_ref, xp_ref, wg_ref, bg_ref, ag_ref, S_ref):
        full = jnp.concatenate([s_ref[c] + hp_ref[c] for c in range(4)],
                               axis=1)
        ag = full * xp_ref[:, 2:3]
        xg = lax.dot_general(ag.astype(jnp.bfloat16), wg_ref[...],
                             (((1,), (0,)), ((), ())),
                             preferred_element_type=f32) + bg_ref[...]
        ag_ref[...] = xg
        S_ref[...] = jnp.sum(xg.reshape(R // NPG, NPG, H), axis=1,
                             keepdims=True)

    return pl.pallas_call(
        body,
        grid=(N // R,),
        in_specs=[pl.BlockSpec((4, R, 32), lambda i: (0, i, 0)),
                  pl.BlockSpec((4, R, 32), lambda i: (0, i, 0)),
                  pl.BlockSpec((R, 16), lambda i: (i, 0)),
                  pl.BlockSpec((H, H), lambda i: (0, 0)),
                  pl.BlockSpec((1, H), lambda i: (0, 0))],
        out_specs=[pl.BlockSpec((R, H), lambda i: (i, 0)),
                   pl.BlockSpec((R // NPG, 1, H), lambda i: (i, 0, 0))],
        out_shape=[jax.ShapeDtypeStruct((N, H), f32),
                   jax.ShapeDtypeStruct((B, 1, H), f32)],
    )(scat2, hp, xp, W_g2.astype(jnp.bfloat16), b_g2.reshape(1, H))


def tc_main(x12, S, ns3, invd, incp, incc, A, AT, Ws1r, Wc1r,
            W_s2, b_s2, W_c2, b_c2, bs1, bc1, xiT):
    GB = 8
    MB = GB * M

    def body(x12_ref, S_ref, ns_ref, invd_ref, incp_ref, incc_ref,
             A_ref, AT_ref, Gs_ref, Gc_ref, Ws2_ref, bs2_ref, Wc2_ref,
             bc2_ref, bse_ref, bce_ref, xiT_ref, z_ref, zc_ref):
        dot = lambda a, b: lax.dot_general(
            a, b, (((1,), (0,)), ((), ())), preferred_element_type=f32)
        dotT = lambda a, b: lax.dot_general(
            a, b, (((1,), (1,)), ((), ())), preferred_element_type=f32)
        bf = jnp.bfloat16
        x1 = x12_ref[0].astype(bf)
        x2 = x12_ref[1].astype(bf)
        Sg = S_ref[:, 0, :].astype(bf)
        sg_term = dot(Sg, Gs_ref[2]) + bse_ref[...]
        cg_term = dot(Sg, Gc_ref[2]) + bce_ref[...]
        sg_full = jnp.repeat(sg_term, M, axis=0)
        cg_full = jnp.repeat(cg_term, M, axis=0)
        s_pre = dot(x1, Gs_ref[0]) + dot(x2, Gs_ref[1]) + sg_full
        sml = dot(jax.nn.relu(s_pre).astype(bf), Ws2_ref[...]) + bs2_ref[...]
        c_pre = dot(x1, Gc_ref[0]) + dot(x2, Gc_ref[1]) + cg_full
        cml = dot(jax.nn.relu(c_pre).astype(bf), Wc2_ref[...]) + bc2_ref[...]
        smlT = sml.T.reshape(4, GB, M)
        cmlT = cml.T.reshape(3, GB, M)
        ns = ns_ref[:, 0, :]
        jidx = lax.broadcasted_iota(i32, (GB, M), 1)
        mask = jidx >= (M - ns)
        one = jnp.ones((GB, M), f32)
        zero = jnp.zeros((GB, M), f32)
        graph_topo = jnp.where(mask, jax.nn.sigmoid(smlT[0]), one)
        p_flow = (jnp.where(mask, smlT[1], zero)
                  + jnp.where(mask, zero, cmlT[0]))
        vp = jnp.where(mask, smlT[2], zero) + jnp.where(mask, zero, cmlT[1])
        vcv = jnp.where(mask, smlT[3], zero) + jnp.where(mask, zero, cmlT[2])
        vrows = [dotT(vp[g:g + 1], incp_ref[g]) + dotT(vcv[g:g + 1],
                 incc_ref[g]) for g in range(GB)]
        vsum = jnp.concatenate(vrows, axis=0)
        v = invd_ref[:, 0, :] * vsum
        lidx = lax.broadcasted_iota(i32, (GB, NPG), 1)
        v = jnp.where(lidx == 0, jnp.float32(1.0), v)
        pfc = p_flow * graph_topo
        qfc = dot(v, A_ref[...]) * graph_topo
        pg = xiT_ref[:, 0, :] + dot(pfc, AT_ref[...])
        qg = xiT_ref[:, 1, :] + dot(qfc, AT_ref[...])
        z_ref[:, 0, 0:M] = pfc
        z_ref[:, 0, M:M + NPG] = v
        z_ref[:, 0, M + NPG:M + NPG + M] = graph_topo
        zc_ref[:, 0, 0:M] = qfc
        zc_ref[:, 0, M:M + NPG] = pg
        zc_ref[:, 0, M + NPG:M + 2 * NPG] = qg

    return pl.pallas_call(
        body,
        grid=(B // GB,),
        in_specs=[pl.BlockSpec((2, MB, H), lambda g: (0, g, 0)),
                  pl.BlockSpec((GB, 1, H), lambda g: (g, 0, 0)),
                  pl.BlockSpec((GB, 1, 1), lambda g: (g, 0, 0)),
                  pl.BlockSpec((GB, 1, NPG), lambda g: (g, 0, 0)),
                  pl.BlockSpec((GB, NPG, M), lambda g: (g, 0, 0)),
                  pl.BlockSpec((GB, NPG, M), lambda g: (g, 0, 0)),
                  pl.BlockSpec((NPG, M), lambda g: (0, 0)),
                  pl.BlockSpec((M, NPG), lambda g: (0, 0)),
                  pl.BlockSpec((3, H, 4 * H), lambda g: (0, 0, 0)),
                  pl.BlockSpec((3, H, 3 * H), lambda g: (0, 0, 0)),
                  pl.BlockSpec((4 * H, 4), lambda g: (0, 0)),
                  pl.BlockSpec((1, 4), lambda g: (0, 0)),
                  pl.BlockSpec((3 * H, 3), lambda g: (0, 0)),
                  pl.BlockSpec((1, 3), lambda g: (0, 0)),
                  pl.BlockSpec((1, 4 * H), lambda g: (0, 0)),
                  pl.BlockSpec((1, 3 * H), lambda g: (0, 0)),
                  pl.BlockSpec((GB, 2, NPG), lambda g: (g, 0, 0))],
        out_specs=[pl.BlockSpec((GB, 1, 2 * M + NPG), lambda g: (g, 0, 0)),
                   pl.BlockSpec((GB, 1, M + 2 * NPG), lambda g: (g, 0, 0))],
        out_shape=[jax.ShapeDtypeStruct((B, 1, 2 * M + NPG), f32),
                   jax.ShapeDtypeStruct((B, 1, M + 2 * NPG), f32)],
    )(x12, S, ns3, invd, incp, incc, A, AT, Ws1r, Wc1r,
      W_s2, b_s2.reshape(1, 4), W_c2, b_c2.reshape(1, 3), bs1, bc1,
      xiT)


def kernel(x, edge_index, numSwitches, inv_degree, inc_parents, inc_childs, A,
           W_g1, b_g1, W_g2, b_g2, W_s1, b_s1, W_s2, b_s2, W_c1, b_c1,
           W_c2, b_c2):
    e0 = edge_index[0].astype(i32)
    e1 = edge_index[1].astype(i32)
    s12 = e0.reshape(2, 16, 40, KB)
    d12 = e1.reshape(2, 16, 40, KB)
    d3 = e1.reshape(16, 80, KB)
    s3 = e0.reshape(16, 80, KB)
    er = e0.reshape(B, 2 * M)
    pcidx = jnp.stack([er[:, :M].reshape(-1),
                       er[:, M:].reshape(-1)]).reshape(2, 16, 40, KB)
    zeros16 = jnp.zeros((STRIPE, 16), f32)
    zeros32 = jnp.zeros((STRIPE, 32), f32)
    ones_src = jnp.ones((KB, 16), f32)

    cnt = sc_count(d12, zeros16, ones_src)
    xp = tc_prep(cnt, x)
    scat1 = sc_scatter16(s12, d12, xp, zeros16)
    hp = tc_conv1(scat1, xp, W_g1, b_g1)
    scat2 = sc_conv2(s3, d3, hp, zeros32)
    ag, S = tc_finalize(scat2.reshape(4, NP, 32), hp, xp, W_g2, b_g2)
    x12 = sc_gather(ag, pcidx)

    ns3 = numSwitches.astype(i32).reshape(B, 1, 1)
    invd = inv_degree.reshape(B, 1, NPG)
    xiT = x.reshape(B, NPG, 2).transpose(0, 2, 1)
    bf = jnp.bfloat16
    z, zc = tc_main(x12, S, ns3, invd, inc_parents, inc_childs, A, A.T,
                    W_s1.reshape(3, H, 4 * H).astype(bf),
                    W_c1.reshape(3, H, 3 * H).astype(bf),
                    W_s2.astype(bf), b_s2, W_c2.astype(bf), b_c2,
                    b_s1.reshape(1, 4 * H), b_c1.reshape(1, 3 * H), xiT)
    return (z.reshape(B, 2 * M + NPG), zc.reshape(B, M + 2 * NPG))
